# baseline (speedup 1.0000x reference)
import functools

import jax
import jax.numpy as jnp
from jax import lax
from jax.experimental import pallas as pl
from jax.experimental.pallas import tpu as pltpu
from jax.experimental.pallas import tpu_sc as plsc

B = 16384
F = 26
D = 16
VOCAB = 1000000

NC = 2
NS = 16
NW = NC * NS
COLS = B // NW
GATHERS = F * COLS // 128


def _gather_body(idx_hbm, w_hbm, fo_hbm, idx_v, wv_v, gsem):
    c = lax.axis_index("c")
    s = lax.axis_index("s")
    wid = s * NC + c
    col0 = wid * COLS

    pltpu.sync_copy(idx_hbm.at[:, pl.ds(col0, COLS)], idx_v)

    def fire(j, carry):
        f = j // (COLS // 128)
        k = j % (COLS // 128)
        pltpu.async_copy(
            w_hbm.at[idx_v.at[f, pl.ds(k * 128, 128)]],
            wv_v.at[f, pl.ds(k * 128, 128)],
            gsem,
        )
        return carry

    lax.fori_loop(0, GATHERS, fire, 0)

    def drain(j, carry):
        pltpu.make_async_copy(
            w_hbm.at[pl.ds(0, 128)], wv_v.at[0, pl.ds(0, 128)], gsem
        ).wait()
        return carry

    lax.fori_loop(0, GATHERS, drain, 0)

    pltpu.sync_copy(wv_v, fo_hbm.at[:, pl.ds(col0, COLS)])


@functools.cache
def _gather_sc():
    return pl.kernel(
        _gather_body,
        out_type=jax.ShapeDtypeStruct((F, B), jnp.float32),
        mesh=plsc.VectorSubcoreMesh(
            core_axis_name="c", subcore_axis_name="s",
            num_cores=NC, num_subcores=NS,
        ),
        scratch_types=[
            pltpu.VMEM((F, COLS), jnp.int32),
            pltpu.VMEM((F, COLS), jnp.float32),
            pltpu.SemaphoreType.DMA,
        ],
    )


def _w_copy_body(w_ref, wf_ref, sem):
    c = pltpu.make_async_copy(w_ref.at[0, :], wf_ref, sem)
    c.start()
    c.wait()


def _w_copy(w):
    return pl.pallas_call(
        _w_copy_body,
        in_specs=[pl.BlockSpec(memory_space=pl.ANY)],
        out_specs=pl.BlockSpec(memory_space=pl.ANY),
        out_shape=jax.ShapeDtypeStruct((VOCAB,), jnp.float32),
        scratch_shapes=[pltpu.SemaphoreType.DMA],
    )(w.T)


def _pool_body(e_ref, o_ref):
    e = e_ref[...]
    ssum = jnp.sum(e, axis=0)
    ssq = jnp.sum(e * e, axis=0)
    o_ref[...] = 0.5 * (ssum * ssum - ssq)


def _pool_tc(eT):
    blk = 2048
    return pl.pallas_call(
        _pool_body,
        grid=(B // blk,),
        in_specs=[pl.BlockSpec((F, D, blk), lambda i: (0, 0, i))],
        out_specs=pl.BlockSpec((D, blk), lambda i: (0, i)),
        out_shape=jax.ShapeDtypeStruct((D, B), jnp.float32),
    )(eT)


def kernel(sparse_inputs, embed_inputs, w):
    idxT = sparse_inputs.T
    eT = embed_inputs.transpose(1, 2, 0)
    wf = _w_copy(w)
    foT = _gather_sc()(idxT, wf)
    soT = _pool_tc(eT)
    return jnp.concatenate([foT, soT], axis=0).T


# speedup vs baseline: 3.4933x; 3.4933x over previous
import functools

import jax
import jax.numpy as jnp
from jax import lax
from jax.experimental import pallas as pl
from jax.experimental.pallas import tpu as pltpu
from jax.experimental.pallas import tpu_sc as plsc

B = 16384
F = 26
D = 16
VOCAB = 1000000

NC = 2
NS = 16
NW = NC * NS
COLS = B // NW
GATHERS = F * COLS // 128


WPAD = VOCAB + 448


def _gather_body(idx_hbm, w_hbm, fo_hbm, idx_v, wv_v, w_bounce, w_sh, gsem):
    c = lax.axis_index("c")
    s = lax.axis_index("s")
    wid = s * NC + c
    col0 = wid * COLS

    seg = WPAD // NS
    chunk = F * COLS
    for i in range(5):
        n = chunk if i < 4 else seg - 4 * chunk
        off = s * seg + i * chunk
        pltpu.sync_copy(w_hbm.at[pl.ds(off, n)], w_bounce.at[pl.ds(0, n)])
        pltpu.sync_copy(w_bounce.at[pl.ds(0, n)], w_sh.at[pl.ds(off, n)])
    pltpu.sync_copy(idx_hbm.at[:, pl.ds(col0, COLS)], idx_v)
    plsc.subcore_barrier()

    def fire(j, carry):
        f = j // (COLS // 128)
        k = j % (COLS // 128)
        pltpu.async_copy(
            w_sh.at[idx_v.at[f, pl.ds(k * 128, 128)]],
            wv_v.at[f, pl.ds(k * 128, 128)],
            gsem,
        )
        return carry

    lax.fori_loop(0, GATHERS, fire, 0)

    def drain(j, carry):
        pltpu.make_async_copy(
            w_hbm.at[pl.ds(0, 128)], wv_v.at[0, pl.ds(0, 128)], gsem
        ).wait()
        return carry

    lax.fori_loop(0, GATHERS, drain, 0)

    pltpu.sync_copy(wv_v, fo_hbm.at[:, pl.ds(col0, COLS)])


@functools.cache
def _gather_sc():
    return pl.kernel(
        _gather_body,
        out_type=jax.ShapeDtypeStruct((F, B), jnp.float32),
        mesh=plsc.VectorSubcoreMesh(
            core_axis_name="c", subcore_axis_name="s",
            num_cores=NC, num_subcores=NS,
        ),
        scratch_types=[
            pltpu.VMEM((F, COLS), jnp.int32),
            pltpu.VMEM((F, COLS), jnp.float32),
            pltpu.VMEM((F * COLS,), jnp.float32),
            pltpu.VMEM_SHARED((WPAD,), jnp.float32),
            pltpu.SemaphoreType.DMA,
        ],
    )


def _pool_body(e_ref, o_ref):
    e = e_ref[...]
    ssum = jnp.sum(e, axis=0)
    ssq = jnp.sum(e * e, axis=0)
    o_ref[...] = 0.5 * (ssum * ssum - ssq)


def _pool_tc(eT):
    blk = 2048
    return pl.pallas_call(
        _pool_body,
        grid=(B // blk,),
        in_specs=[pl.BlockSpec((F, D, blk), lambda i: (0, 0, i))],
        out_specs=pl.BlockSpec((D, blk), lambda i: (0, i)),
        out_shape=jax.ShapeDtypeStruct((D, B), jnp.float32),
    )(eT)


def kernel(sparse_inputs, embed_inputs, w):
    idxT = sparse_inputs.T
    eT = embed_inputs.transpose(1, 2, 0)
    wf = jnp.pad(w, ((0, 448), (0, 0))).reshape(VOCAB + 448)
    foT = _gather_sc()(idxT, wf)
    soT = _pool_tc(eT)
    return jnp.concatenate([foT, soT], axis=0).T


# speedup vs baseline: 3.6614x; 1.0481x over previous
import functools

import jax
import jax.numpy as jnp
from jax import lax
from jax.experimental import pallas as pl
from jax.experimental.pallas import tpu as pltpu
from jax.experimental.pallas import tpu_sc as plsc

B = 16384
F = 26
D = 16
VOCAB = 1000000

NC = 2
NS = 16
NW = NC * NS
COLS = B // NW
GATHERS = F * COLS // 128


WPAD = VOCAB + 448


def _gather_body(
    idx_hbm, w_hbm, fo_hbm, idx_v, wv_v, b0, b1, w_sh, gsem, hsem, ssem, isem
):
    c = lax.axis_index("c")
    s = lax.axis_index("s")
    wid = s * NC + c
    col0 = wid * COLS

    idx_copy = pltpu.async_copy(idx_hbm.at[:, pl.ds(col0, COLS)], idx_v, isem)

    seg = WPAD // NS
    chunk = F * COLS
    bounce = (b0, b1)
    sp_copies = [None, None]
    for i in range(5):
        n = chunk if i < 4 else seg - 4 * chunk
        off = s * seg + i * chunk
        bi = i % 2
        if sp_copies[bi] is not None:
            sp_copies[bi].wait()
        pltpu.async_copy(
            w_hbm.at[pl.ds(off, n)], bounce[bi].at[pl.ds(0, n)], hsem
        ).wait()
        sp_copies[bi] = pltpu.async_copy(
            bounce[bi].at[pl.ds(0, n)], w_sh.at[pl.ds(off, n)], ssem
        )
    sp_copies[0].wait()
    sp_copies[1].wait()
    idx_copy.wait()
    plsc.subcore_barrier()

    def fire(j, carry):
        f = j // (COLS // 128)
        k = j % (COLS // 128)
        pltpu.async_copy(
            w_sh.at[idx_v.at[f, pl.ds(k * 128, 128)]],
            wv_v.at[f, pl.ds(k * 128, 128)],
            gsem,
        )
        return carry

    lax.fori_loop(0, GATHERS, fire, 0)

    def drain(j, carry):
        pltpu.make_async_copy(
            w_hbm.at[pl.ds(0, 128)], wv_v.at[0, pl.ds(0, 128)], gsem
        ).wait()
        return carry

    lax.fori_loop(0, GATHERS, drain, 0)

    pltpu.sync_copy(wv_v, fo_hbm.at[:, pl.ds(col0, COLS)])


@functools.cache
def _gather_sc():
    return pl.kernel(
        _gather_body,
        out_type=jax.ShapeDtypeStruct((F, B), jnp.float32),
        mesh=plsc.VectorSubcoreMesh(
            core_axis_name="c", subcore_axis_name="s",
            num_cores=NC, num_subcores=NS,
        ),
        scratch_types=[
            pltpu.VMEM((F, COLS), jnp.int32),
            pltpu.VMEM((F, COLS), jnp.float32),
            pltpu.VMEM((F * COLS,), jnp.float32),
            pltpu.VMEM((F * COLS,), jnp.float32),
            pltpu.VMEM_SHARED((WPAD,), jnp.float32),
            pltpu.SemaphoreType.DMA,
            pltpu.SemaphoreType.DMA,
            pltpu.SemaphoreType.DMA,
            pltpu.SemaphoreType.DMA,
        ],
    )


def _pool_body(e_ref, o_ref):
    e = e_ref[...]
    ssum = jnp.sum(e, axis=0)
    ssq = jnp.sum(e * e, axis=0)
    o_ref[...] = 0.5 * (ssum * ssum - ssq)


def _pool_tc(eT):
    blk = 2048
    return pl.pallas_call(
        _pool_body,
        grid=(B // blk,),
        in_specs=[pl.BlockSpec((F, D, blk), lambda i: (0, 0, i))],
        out_specs=pl.BlockSpec((D, blk), lambda i: (0, i)),
        out_shape=jax.ShapeDtypeStruct((D, B), jnp.float32),
    )(eT)


def kernel(sparse_inputs, embed_inputs, w):
    idxT = sparse_inputs.T
    eT = embed_inputs.transpose(1, 2, 0)
    wf = jnp.pad(w, ((0, 448), (0, 0))).reshape(VOCAB + 448)
    foT = _gather_sc()(idxT, wf)
    soT = _pool_tc(eT)
    return jnp.concatenate([foT, soT], axis=0).T
